# baseline (device time: 1416718 ns/iter reference)
import jax
import jax.numpy as jnp
from jax import lax
from jax.experimental import pallas as pl
from jax.experimental.pallas import tpu as pltpu

N_DEV = 4
NC = 4
N_RS = (N_DEV - 1) * NC
N_IT = 2 * N_RS


def _gelu(y):
    c = 0.7978845608028654
    return 0.5 * y * (1.0 + jnp.tanh(c * (y + 0.044715 * y * y * y)))


def kernel(x, w_mat):
    partial = jnp.dot(x, w_mat, preferred_element_type=jnp.float32)
    return _allreduce_gelu(partial)


def _allreduce_gelu(partial):
    M, N = partial.shape
    HM = M // 2
    BLK = HM // N_DEV
    CH = BLK // NC

    def body(p_ref, out_ref, acc_r_hbm, acc_l_hbm, recv_r_hbm, recv_l_hbm,
             a_buf, b_buf,
             send_sems_r, recv_sems_r, send_sems_l, recv_sems_l,
             credit_r, credit_l, cp_sems, wb_sems):
        my = lax.axis_index("i")
        left = (my + N_DEV - 1) % N_DEV
        right = (my + 1) % N_DEV

        barrier = pltpu.get_barrier_semaphore()
        for nbr in (left, right):
            pl.semaphore_signal(barrier, inc=1, device_id=(nbr,),
                                device_id_type=pl.DeviceIdType.MESH)
        pl.semaphore_wait(barrier, 2)

        def make_rdma(it):
            slot = it % 2
            if it < N_RS:
                h, c = divmod(it, NC)
                ro = c * CH
                sb_r = (my + N_DEV - h) % N_DEV
                sb_l = (my + h) % N_DEV
                if h == 0:
                    src_r = p_ref.at[pl.ds(sb_r * BLK + ro, CH), :]
                    src_l = p_ref.at[pl.ds(HM + sb_l * BLK + ro, CH), :]
                else:
                    src_r = acc_r_hbm.at[(h - 1) % 2, pl.ds(ro, CH), :]
                    src_l = acc_l_hbm.at[(h - 1) % 2, pl.ds(ro, CH), :]
                dst_r = recv_r_hbm.at[slot]
                dst_l = recv_l_hbm.at[slot]
            else:
                g, c = divmod(it - N_RS, NC)
                ro = c * CH
                gs_r = (my + N_DEV + 1 - g) % N_DEV
                gs_l = (my + N_DEV - 1 + g) % N_DEV
                src_r = dst_r = out_ref.at[pl.ds(gs_r * BLK + ro, CH), :]
                src_l = dst_l = out_ref.at[pl.ds(HM + gs_l * BLK + ro, CH), :]
            rdma_r = pltpu.make_async_remote_copy(
                src_ref=src_r, dst_ref=dst_r,
                send_sem=send_sems_r.at[slot], recv_sem=recv_sems_r.at[slot],
                device_id=(right,), device_id_type=pl.DeviceIdType.MESH)
            rdma_l = pltpu.make_async_remote_copy(
                src_ref=src_l, dst_ref=dst_l,
                send_sem=send_sems_l.at[slot], recv_sem=recv_sems_l.at[slot],
                device_id=(left,), device_id_type=pl.DeviceIdType.MESH)
            return rdma_r, rdma_l

        descs = {}

        def start(it):
            if it >= 2:
                descs[it - 2][0].wait_send()
                descs[it - 2][1].wait_send()
                pl.semaphore_wait(credit_r, 1)
                pl.semaphore_wait(credit_l, 1)
            d = make_rdma(it)
            descs[it] = d
            d[0].start()
            d[1].start()

        def consume(it):
            slot = it % 2
            if it < N_RS:
                h, c = divmod(it, NC)
                ro = c * CH
                rb_r = (my + N_DEV - h - 1) % N_DEV
                rb_l = (my + h + 1) % N_DEV
                cps = [
                    pltpu.make_async_copy(
                        recv_r_hbm.at[slot], a_buf.at[pl.ds(0, CH), :],
                        cp_sems.at[0]),
                    pltpu.make_async_copy(
                        recv_l_hbm.at[slot], a_buf.at[pl.ds(CH, CH), :],
                        cp_sems.at[1]),
                    pltpu.make_async_copy(
                        p_ref.at[pl.ds(rb_r * BLK + ro, CH), :],
                        b_buf.at[pl.ds(0, CH), :], cp_sems.at[2]),
                    pltpu.make_async_copy(
                        p_ref.at[pl.ds(HM + rb_l * BLK + ro, CH), :],
                        b_buf.at[pl.ds(CH, CH), :], cp_sems.at[3]),
                ]
                for cp in cps:
                    cp.start()
                for cp in cps:
                    cp.wait()
                pl.semaphore_signal(credit_r, inc=1, device_id=(left,),
                                    device_id_type=pl.DeviceIdType.MESH)
                pl.semaphore_signal(credit_l, inc=1, device_id=(right,),
                                    device_id_type=pl.DeviceIdType.MESH)
                if h < N_DEV - 2:
                    a_buf[:, :] = a_buf[:, :] + b_buf[:, :]
                    wbs = [
                        pltpu.make_async_copy(
                            a_buf.at[pl.ds(0, CH), :],
                            acc_r_hbm.at[h % 2, pl.ds(ro, CH), :],
                            wb_sems.at[0]),
                        pltpu.make_async_copy(
                            a_buf.at[pl.ds(CH, CH), :],
                            acc_l_hbm.at[h % 2, pl.ds(ro, CH), :],
                            wb_sems.at[1]),
                    ]
                else:
                    a_buf[:, :] = _gelu(a_buf[:, :] + b_buf[:, :])
                    wbs = [
                        pltpu.make_async_copy(
                            a_buf.at[pl.ds(0, CH), :],
                            out_ref.at[pl.ds(rb_r * BLK + ro, CH), :],
                            wb_sems.at[0]),
                        pltpu.make_async_copy(
                            a_buf.at[pl.ds(CH, CH), :],
                            out_ref.at[pl.ds(HM + rb_l * BLK + ro, CH), :],
                            wb_sems.at[1]),
                    ]
                for wb in wbs:
                    wb.start()
                for wb in wbs:
                    wb.wait()
            else:
                pl.semaphore_signal(credit_r, inc=1, device_id=(left,),
                                    device_id_type=pl.DeviceIdType.MESH)
                pl.semaphore_signal(credit_l, inc=1, device_id=(right,),
                                    device_id_type=pl.DeviceIdType.MESH)

        start(0)
        for it in range(N_IT):
            descs[it][0].wait_recv()
            descs[it][1].wait_recv()
            if it + 1 < N_IT:
                start(it + 1)
            consume(it)

        for it in (N_IT - 2, N_IT - 1):
            descs[it][0].wait_send()
            descs[it][1].wait_send()
        pl.semaphore_wait(credit_r, 2)
        pl.semaphore_wait(credit_l, 2)

    out, *_ = pl.pallas_call(
        body,
        out_shape=[
            jax.ShapeDtypeStruct((M, N), jnp.float32),
            jax.ShapeDtypeStruct((2, BLK, N), jnp.float32),
            jax.ShapeDtypeStruct((2, BLK, N), jnp.float32),
            jax.ShapeDtypeStruct((2, CH, N), jnp.float32),
            jax.ShapeDtypeStruct((2, CH, N), jnp.float32),
        ],
        in_specs=[pl.BlockSpec(memory_space=pl.ANY)],
        out_specs=[pl.BlockSpec(memory_space=pl.ANY)] * 5,
        scratch_shapes=[
            pltpu.VMEM((2 * CH, N), jnp.float32),
            pltpu.VMEM((2 * CH, N), jnp.float32),
            pltpu.SemaphoreType.DMA((2,)),
            pltpu.SemaphoreType.DMA((2,)),
            pltpu.SemaphoreType.DMA((2,)),
            pltpu.SemaphoreType.DMA((2,)),
            pltpu.SemaphoreType.REGULAR,
            pltpu.SemaphoreType.REGULAR,
            pltpu.SemaphoreType.DMA((4,)),
            pltpu.SemaphoreType.DMA((2,)),
        ],
        compiler_params=pltpu.CompilerParams(collective_id=0),
    )(partial)
    return out


# device time: 1363480 ns/iter; 1.0390x vs baseline; 1.0390x over previous
import jax
import jax.numpy as jnp
from jax import lax
from jax.experimental import pallas as pl
from jax.experimental.pallas import tpu as pltpu

N_DEV = 4
NC = 4
N_RS = (N_DEV - 1) * NC
N_IT = 2 * N_RS
N_SLOT = 4
LOOK = 3


def _gelu(y):
    c = 0.7978845608028654
    return 0.5 * y * (1.0 + jnp.tanh(c * (y + 0.044715 * y * y * y)))


def kernel(x, w_mat):
    partial = jnp.dot(x, w_mat, preferred_element_type=jnp.float32)
    return _allreduce_gelu(partial)


def _allreduce_gelu(partial):
    M, N = partial.shape
    HM = M // 2
    BLK = HM // N_DEV
    CH = BLK // NC

    def body(p_ref, out_ref, acc_r_hbm, acc_l_hbm, recv_r_hbm, recv_l_hbm,
             a_buf, b_buf,
             send_sems_r, recv_sems_r, send_sems_l, recv_sems_l,
             credit_r, credit_l, cp_sems, wb_sems):
        my = lax.axis_index("i")
        left = (my + N_DEV - 1) % N_DEV
        right = (my + 1) % N_DEV

        barrier = pltpu.get_barrier_semaphore()
        for nbr in (left, right):
            pl.semaphore_signal(barrier, inc=1, device_id=(nbr,),
                                device_id_type=pl.DeviceIdType.MESH)
        pl.semaphore_wait(barrier, 2)

        def make_rdma(it):
            slot = it % N_SLOT
            if it < N_RS:
                h, c = divmod(it, NC)
                ro = c * CH
                sb_r = (my + N_DEV - h) % N_DEV
                sb_l = (my + h) % N_DEV
                if h == 0:
                    src_r = p_ref.at[pl.ds(sb_r * BLK + ro, CH), :]
                    src_l = p_ref.at[pl.ds(HM + sb_l * BLK + ro, CH), :]
                else:
                    src_r = acc_r_hbm.at[(h - 1) % 2, pl.ds(ro, CH), :]
                    src_l = acc_l_hbm.at[(h - 1) % 2, pl.ds(ro, CH), :]
                dst_r = recv_r_hbm.at[slot]
                dst_l = recv_l_hbm.at[slot]
            else:
                g, c = divmod(it - N_RS, NC)
                ro = c * CH
                gs_r = (my + N_DEV + 1 - g) % N_DEV
                gs_l = (my + N_DEV - 1 + g) % N_DEV
                src_r = dst_r = out_ref.at[pl.ds(gs_r * BLK + ro, CH), :]
                src_l = dst_l = out_ref.at[pl.ds(HM + gs_l * BLK + ro, CH), :]
            rdma_r = pltpu.make_async_remote_copy(
                src_ref=src_r, dst_ref=dst_r,
                send_sem=send_sems_r.at[slot], recv_sem=recv_sems_r.at[slot],
                device_id=(right,), device_id_type=pl.DeviceIdType.MESH)
            rdma_l = pltpu.make_async_remote_copy(
                src_ref=src_l, dst_ref=dst_l,
                send_sem=send_sems_l.at[slot], recv_sem=recv_sems_l.at[slot],
                device_id=(left,), device_id_type=pl.DeviceIdType.MESH)
            return rdma_r, rdma_l

        descs = {}

        def start(it):
            if it >= N_SLOT:
                descs[it - N_SLOT][0].wait_send()
                descs[it - N_SLOT][1].wait_send()
                pl.semaphore_wait(credit_r, 1)
                pl.semaphore_wait(credit_l, 1)
            d = make_rdma(it)
            descs[it] = d
            d[0].start()
            d[1].start()

        def consume(it):
            slot = it % N_SLOT
            if it < N_RS:
                h, c = divmod(it, NC)
                ro = c * CH
                rb_r = (my + N_DEV - h - 1) % N_DEV
                rb_l = (my + h + 1) % N_DEV
                cps = [
                    pltpu.make_async_copy(
                        recv_r_hbm.at[slot], a_buf.at[pl.ds(0, CH), :],
                        cp_sems.at[0]),
                    pltpu.make_async_copy(
                        recv_l_hbm.at[slot], a_buf.at[pl.ds(CH, CH), :],
                        cp_sems.at[1]),
                    pltpu.make_async_copy(
                        p_ref.at[pl.ds(rb_r * BLK + ro, CH), :],
                        b_buf.at[pl.ds(0, CH), :], cp_sems.at[2]),
                    pltpu.make_async_copy(
                        p_ref.at[pl.ds(HM + rb_l * BLK + ro, CH), :],
                        b_buf.at[pl.ds(CH, CH), :], cp_sems.at[3]),
                ]
                for cp in cps:
                    cp.start()
                for cp in cps:
                    cp.wait()
                pl.semaphore_signal(credit_r, inc=1, device_id=(left,),
                                    device_id_type=pl.DeviceIdType.MESH)
                pl.semaphore_signal(credit_l, inc=1, device_id=(right,),
                                    device_id_type=pl.DeviceIdType.MESH)
                if h < N_DEV - 2:
                    a_buf[:, :] = a_buf[:, :] + b_buf[:, :]
                    wbs = [
                        pltpu.make_async_copy(
                            a_buf.at[pl.ds(0, CH), :],
                            acc_r_hbm.at[h % 2, pl.ds(ro, CH), :],
                            wb_sems.at[0]),
                        pltpu.make_async_copy(
                            a_buf.at[pl.ds(CH, CH), :],
                            acc_l_hbm.at[h % 2, pl.ds(ro, CH), :],
                            wb_sems.at[1]),
                    ]
                else:
                    a_buf[:, :] = _gelu(a_buf[:, :] + b_buf[:, :])
                    wbs = [
                        pltpu.make_async_copy(
                            a_buf.at[pl.ds(0, CH), :],
                            out_ref.at[pl.ds(rb_r * BLK + ro, CH), :],
                            wb_sems.at[0]),
                        pltpu.make_async_copy(
                            a_buf.at[pl.ds(CH, CH), :],
                            out_ref.at[pl.ds(HM + rb_l * BLK + ro, CH), :],
                            wb_sems.at[1]),
                    ]
                for wb in wbs:
                    wb.start()
                for wb in wbs:
                    wb.wait()
            else:
                pl.semaphore_signal(credit_r, inc=1, device_id=(left,),
                                    device_id_type=pl.DeviceIdType.MESH)
                pl.semaphore_signal(credit_l, inc=1, device_id=(right,),
                                    device_id_type=pl.DeviceIdType.MESH)

        for it in range(LOOK):
            start(it)
        for it in range(N_IT):
            descs[it][0].wait_recv()
            descs[it][1].wait_recv()
            if it + LOOK < N_IT:
                start(it + LOOK)
            consume(it)

        for it in range(N_IT - N_SLOT, N_IT):
            descs[it][0].wait_send()
            descs[it][1].wait_send()
        pl.semaphore_wait(credit_r, N_SLOT)
        pl.semaphore_wait(credit_l, N_SLOT)

    out, *_ = pl.pallas_call(
        body,
        out_shape=[
            jax.ShapeDtypeStruct((M, N), jnp.float32),
            jax.ShapeDtypeStruct((2, BLK, N), jnp.float32),
            jax.ShapeDtypeStruct((2, BLK, N), jnp.float32),
            jax.ShapeDtypeStruct((N_SLOT, CH, N), jnp.float32),
            jax.ShapeDtypeStruct((N_SLOT, CH, N), jnp.float32),
        ],
        in_specs=[pl.BlockSpec(memory_space=pl.ANY)],
        out_specs=[pl.BlockSpec(memory_space=pl.ANY)] * 5,
        scratch_shapes=[
            pltpu.VMEM((2 * CH, N), jnp.float32),
            pltpu.VMEM((2 * CH, N), jnp.float32),
            pltpu.SemaphoreType.DMA((N_SLOT,)),
            pltpu.SemaphoreType.DMA((N_SLOT,)),
            pltpu.SemaphoreType.DMA((N_SLOT,)),
            pltpu.SemaphoreType.DMA((N_SLOT,)),
            pltpu.SemaphoreType.REGULAR,
            pltpu.SemaphoreType.REGULAR,
            pltpu.SemaphoreType.DMA((4,)),
            pltpu.SemaphoreType.DMA((2,)),
        ],
        compiler_params=pltpu.CompilerParams(collective_id=0),
    )(partial)
    return out
